# Initial kernel scaffold; baseline (speedup 1.0000x reference)
import functools

import jax
import jax.numpy as jnp
from jax import lax
from jax.experimental import pallas as pl
from jax.experimental.pallas import tpu as pltpu
from jax.experimental.pallas import tpu_sc as plsc

HIDDEN = 64
_SC_CHUNK = 1000


def _make_sc_gather(n_rows, d):
    info = plsc.get_sparse_core_info()
    nw = info.num_cores * info.num_subcores
    assert n_rows % nw == 0
    per_w = n_rows // nw
    chunk = min(_SC_CHUNK, per_w)
    assert per_w % chunk == 0 and chunk % 8 == 0
    n_iter = per_w // chunk
    mesh = plsc.VectorSubcoreMesh(core_axis_name="c", subcore_axis_name="s")

    @functools.partial(
        pl.kernel,
        mesh=mesh,
        out_type=jax.ShapeDtypeStruct((n_rows, d), jnp.float32),
        scratch_types=[
            pltpu.VMEM((chunk,), jnp.int32),
            pltpu.VMEM((chunk, d), jnp.float32),
            pltpu.SemaphoreType.DMA,
        ],
    )
    def sc_gather(idx_hbm, table_hbm, out_hbm, idx_v, rows_v, sem):
        wid = lax.axis_index("s") * info.num_cores + lax.axis_index("c")
        base = wid * per_w

        def body(t, carry):
            off = base + t * chunk
            pltpu.sync_copy(idx_hbm.at[pl.ds(off, chunk)], idx_v)
            pltpu.async_copy(table_hbm.at[idx_v], rows_v, sem).wait()
            pltpu.sync_copy(rows_v, out_hbm.at[pl.ds(off, chunk)])
            return carry

        lax.fori_loop(0, n_iter, body, 0)

    return sc_gather


def _tc_ln_kernel(g_ref, seg_ref, pos_ref, segtab_ref, scale_ref, bias_ref,
                  out_ref, *, rows, period):
    x = g_ref[...]
    pos = pos_ref[...]
    x = x.reshape(rows // period, period, HIDDEN) + pos[None, :, :]
    x = x.reshape(rows, HIDDEN)
    s0 = segtab_ref[0, :]
    s1 = segtab_ref[1, :]
    f = seg_ref[...].astype(jnp.float32)
    x = x + s0[None, :] + f * (s1 - s0)[None, :]
    mean = jnp.mean(x, axis=-1, keepdims=True)
    xc = x - mean
    var = jnp.mean(xc * xc, axis=-1, keepdims=True)
    inv = lax.rsqrt(var + 1e-5)
    out_ref[...] = xc * inv * scale_ref[0, :][None, :] + bias_ref[0, :][None, :]


def kernel(inputs, segments, token_table, segment_table, pos_table,
           ln_scale, ln_bias):
    b, s, _ = inputs.shape
    n = b * s * s
    idx_flat = inputs.reshape(n).astype(jnp.int32)
    seg_flat = segments.reshape(n, 1).astype(jnp.int32)

    gathered = _make_sc_gather(n, HIDDEN)(idx_flat, token_table)

    rows = 2000
    assert n % rows == 0 and rows % s == 0
    grid = n // rows
    out = pl.pallas_call(
        functools.partial(_tc_ln_kernel, rows=rows, period=s),
        grid=(grid,),
        in_specs=[
            pl.BlockSpec((rows, HIDDEN), lambda i: (i, 0)),
            pl.BlockSpec((rows, 1), lambda i: (i, 0)),
            pl.BlockSpec((s, HIDDEN), lambda i: (0, 0)),
            pl.BlockSpec((2, HIDDEN), lambda i: (0, 0)),
            pl.BlockSpec((1, HIDDEN), lambda i: (0, 0)),
            pl.BlockSpec((1, HIDDEN), lambda i: (0, 0)),
        ],
        out_specs=pl.BlockSpec((rows, HIDDEN), lambda i: (i, 0)),
        out_shape=jax.ShapeDtypeStruct((n, HIDDEN), jnp.float32),
    )(gathered, seg_flat, pos_table[:s], segment_table,
      ln_scale.reshape(1, HIDDEN), ln_bias.reshape(1, HIDDEN))

    return out.reshape(b, s, s, HIDDEN)


# speedup vs baseline: 3.2997x; 3.2997x over previous
import functools

import jax
import jax.numpy as jnp
from jax import lax
from jax.experimental import pallas as pl
from jax.experimental.pallas import tpu as pltpu
from jax.experimental.pallas import tpu_sc as plsc

HIDDEN = 64
_SC_CHUNK = 1000


def _make_sc_gather(n_rows, d):
    info = plsc.get_sparse_core_info()
    nw = info.num_cores * info.num_subcores
    assert n_rows % nw == 0
    per_w = n_rows // nw
    chunk = min(_SC_CHUNK, per_w)
    assert per_w % chunk == 0 and chunk % 8 == 0
    n_iter = per_w // chunk
    mesh = plsc.VectorSubcoreMesh(core_axis_name="c", subcore_axis_name="s")

    @functools.partial(
        pl.kernel,
        mesh=mesh,
        out_type=jax.ShapeDtypeStruct((n_rows, d), jnp.float32),
        scratch_types=[
            pltpu.VMEM((chunk,), jnp.int32),
            pltpu.VMEM((chunk, d), jnp.float32),
            pltpu.SemaphoreType.DMA,
        ],
        compiler_params=pltpu.CompilerParams(use_tc_tiling_on_sc=False),
    )
    def sc_gather(idx_hbm, table_hbm, out_hbm, idx_v, rows_v, sem):
        wid = lax.axis_index("s") * info.num_cores + lax.axis_index("c")
        base = wid * per_w

        def body(t, carry):
            off = base + t * chunk
            pltpu.sync_copy(idx_hbm.at[pl.ds(off, chunk)], idx_v)
            pltpu.async_copy(table_hbm.at[idx_v], rows_v, sem).wait()
            pltpu.sync_copy(rows_v, out_hbm.at[pl.ds(off, chunk)])
            return carry

        lax.fori_loop(0, n_iter, body, 0)

    return sc_gather


def _tc_ln_kernel(g_ref, seg_ref, pos_ref, segtab_ref, scale_ref, bias_ref,
                  out_ref, *, rows, period):
    x = g_ref[...]
    pos = pos_ref[...]
    x = x.reshape(rows // period, period, HIDDEN) + pos[None, :, :]
    x = x.reshape(rows, HIDDEN)
    s0 = segtab_ref[0, :]
    s1 = segtab_ref[1, :]
    f = seg_ref[...].astype(jnp.float32)
    x = x + s0[None, :] + f * (s1 - s0)[None, :]
    mean = jnp.mean(x, axis=-1, keepdims=True)
    xc = x - mean
    var = jnp.mean(xc * xc, axis=-1, keepdims=True)
    inv = lax.rsqrt(var + 1e-5)
    out_ref[...] = xc * inv * scale_ref[0, :][None, :] + bias_ref[0, :][None, :]


def kernel(inputs, segments, token_table, segment_table, pos_table,
           ln_scale, ln_bias):
    b, s, _ = inputs.shape
    n = b * s * s
    idx_flat = inputs.reshape(n).astype(jnp.int32)
    seg_flat = segments.reshape(n, 1).astype(jnp.int32)

    gathered = _make_sc_gather(n, HIDDEN)(idx_flat, token_table)

    rows = 2000
    assert n % rows == 0 and rows % s == 0
    grid = n // rows
    out = pl.pallas_call(
        functools.partial(_tc_ln_kernel, rows=rows, period=s),
        grid=(grid,),
        in_specs=[
            pl.BlockSpec((rows, HIDDEN), lambda i: (i, 0)),
            pl.BlockSpec((rows, 1), lambda i: (i, 0)),
            pl.BlockSpec((s, HIDDEN), lambda i: (0, 0)),
            pl.BlockSpec((2, HIDDEN), lambda i: (0, 0)),
            pl.BlockSpec((1, HIDDEN), lambda i: (0, 0)),
            pl.BlockSpec((1, HIDDEN), lambda i: (0, 0)),
        ],
        out_specs=pl.BlockSpec((rows, HIDDEN), lambda i: (i, 0)),
        out_shape=jax.ShapeDtypeStruct((n, HIDDEN), jnp.float32),
    )(gathered, seg_flat, pos_table[:s], segment_table,
      ln_scale.reshape(1, HIDDEN), ln_bias.reshape(1, HIDDEN))

    return out.reshape(b, s, s, HIDDEN)
